# baseline (speedup 1.0000x reference)
import jax
import jax.numpy as jnp
from jax.experimental import pallas as pl
from jax.experimental.pallas import tpu as pltpu


def _upsample2_kernel(x_ref, e_ref, o_ref, *, s):
    u = jnp.dot(x_ref[...], e_ref[...], preferred_element_type=o_ref.dtype)
    o_ref[...] = pltpu.repeat(u, s, axis=1)


def _upsample_nearest(x, s):
    N, C, H, W = x.shape
    R = N * C * H
    x2 = x.reshape(R, W)

    sW = s * W
    cols = jnp.arange(sW, dtype=jnp.int32)
    rows = jnp.arange(W, dtype=jnp.int32)
    ew = (cols[None, :] // s == rows[:, None]).astype(x.dtype)

    block_rows = 2048
    grid = (pl.cdiv(R, block_rows),)
    out_w = s * sW

    flops = 2 * R * W * sW
    bytes_accessed = (R * W + R * out_w + W * sW) * x.dtype.itemsize

    out2 = pl.pallas_call(
        lambda xr, er, orr: _upsample2_kernel(xr, er, orr, s=s),
        out_shape=jax.ShapeDtypeStruct((R, out_w), x.dtype),
        grid=grid,
        in_specs=[
            pl.BlockSpec((block_rows, W), lambda g: (g, 0)),
            pl.BlockSpec((W, sW), lambda g: (0, 0)),
        ],
        out_specs=pl.BlockSpec((block_rows, out_w), lambda g: (g, 0)),
        compiler_params=pltpu.CompilerParams(
            dimension_semantics=("parallel",),
            vmem_limit_bytes=48 * 1024 * 1024,
        ),
        cost_estimate=pl.CostEstimate(
            flops=flops, transcendentals=0, bytes_accessed=bytes_accessed),
    )(x2, ew)

    return out2.reshape(N, C, s * H, s * W)


def kernel(x):
    return _upsample_nearest(x, 2)


# speedup vs baseline: 1.0330x; 1.0330x over previous
import jax
import jax.numpy as jnp
from jax.experimental import pallas as pl
from jax.experimental.pallas import tpu as pltpu


def _upsample2_kernel(x_ref, e_ref, o_ref, *, s):
    u = jnp.dot(x_ref[...], e_ref[...], preferred_element_type=o_ref.dtype)
    o_ref[...] = pltpu.repeat(u, s, axis=1)


def _upsample_nearest(x, s):
    N, C, H, W = x.shape
    R = N * C * H
    x2 = x.reshape(R, W)

    sW = s * W
    cols = jnp.arange(sW, dtype=jnp.int32)
    rows = jnp.arange(W, dtype=jnp.int32)
    ew = (cols[None, :] // s == rows[:, None]).astype(x.dtype)

    block_rows = 4096
    grid = (pl.cdiv(R, block_rows),)
    out_w = s * sW

    flops = 2 * R * W * sW
    bytes_accessed = (R * W + R * out_w + W * sW) * x.dtype.itemsize

    out2 = pl.pallas_call(
        lambda xr, er, orr: _upsample2_kernel(xr, er, orr, s=s),
        out_shape=jax.ShapeDtypeStruct((R, out_w), x.dtype),
        grid=grid,
        in_specs=[
            pl.BlockSpec((block_rows, W), lambda g: (g, 0)),
            pl.BlockSpec((W, sW), lambda g: (0, 0)),
        ],
        out_specs=pl.BlockSpec((block_rows, out_w), lambda g: (g, 0)),
        compiler_params=pltpu.CompilerParams(
            dimension_semantics=("parallel",),
            vmem_limit_bytes=48 * 1024 * 1024,
        ),
        cost_estimate=pl.CostEstimate(
            flops=flops, transcendentals=0, bytes_accessed=bytes_accessed),
    )(x2, ew)

    return out2.reshape(N, C, s * H, s * W)


def kernel(x):
    return _upsample_nearest(x, 2)


# speedup vs baseline: 1.0449x; 1.0115x over previous
import jax
import jax.numpy as jnp
from jax.experimental import pallas as pl
from jax.experimental.pallas import tpu as pltpu


def _upsample2_kernel(x_ref, e_ref, o_ref, *, s):
    u = jnp.dot(x_ref[...], e_ref[...], preferred_element_type=o_ref.dtype)
    o_ref[...] = pltpu.repeat(u, s, axis=1)


def _upsample_nearest(x, s):
    N, C, H, W = x.shape
    R = N * C * H
    x2 = x.reshape(R, W)

    sW = s * W
    cols = jnp.arange(sW, dtype=jnp.int32)
    rows = jnp.arange(W, dtype=jnp.int32)
    ew = (cols[None, :] // s == rows[:, None]).astype(x.dtype)

    block_rows = 8192
    grid = (pl.cdiv(R, block_rows),)
    out_w = s * sW

    flops = 2 * R * W * sW
    bytes_accessed = (R * W + R * out_w + W * sW) * x.dtype.itemsize

    out2 = pl.pallas_call(
        lambda xr, er, orr: _upsample2_kernel(xr, er, orr, s=s),
        out_shape=jax.ShapeDtypeStruct((R, out_w), x.dtype),
        grid=grid,
        in_specs=[
            pl.BlockSpec((block_rows, W), lambda g: (g, 0)),
            pl.BlockSpec((W, sW), lambda g: (0, 0)),
        ],
        out_specs=pl.BlockSpec((block_rows, out_w), lambda g: (g, 0)),
        compiler_params=pltpu.CompilerParams(
            dimension_semantics=("parallel",),
            vmem_limit_bytes=48 * 1024 * 1024,
        ),
        cost_estimate=pl.CostEstimate(
            flops=flops, transcendentals=0, bytes_accessed=bytes_accessed),
    )(x2, ew)

    return out2.reshape(N, C, s * H, s * W)


def kernel(x):
    return _upsample_nearest(x, 2)
